# Initial kernel scaffold; baseline (speedup 1.0000x reference)
import jax
import jax.numpy as jnp
from jax.experimental import pallas as pl


def kernel(q, k, v, W, b):
    raise NotImplementedError("write your pallas kernel here")


# speedup vs baseline: 853.5306x; 853.5306x over previous
import functools

import jax
import jax.numpy as jnp
from jax import lax
from jax.experimental import pallas as pl
from jax.experimental.pallas import tpu as pltpu

B, L, H, D = 2, 2048, 16, 64
BLK = 64
NBLK = L // BLK
TK = 4
SCALE = float(D) ** -0.5
CHUNK = 256


def _row_softmax(x):
    m = jnp.max(x, axis=-1, keepdims=True)
    e = jnp.exp(x - m)
    return e / jnp.sum(e, axis=-1, keepdims=True)


def _head_kernel(q_ref, k_ref, v_ref, w_ref, b_ref, out_ref,
                 kh_ref, vh_ref, idx_vmem, idx_smem, sem):
    f32 = jnp.float32
    bf16 = jnp.bfloat16

    kv_acc = jnp.zeros((D, D), dtype=f32)
    ksum = jnp.zeros((1, D), dtype=f32)
    kb_parts = []
    for c in range(L // CHUNK):
        sl = slice(c * CHUNK, (c + 1) * CHUNK)
        kc = k_ref[0, 0, sl, :]
        vc = v_ref[0, 0, sl, :]
        khc = kc.astype(bf16)
        vhc = vc.astype(bf16)
        kh_ref[sl, :] = khc
        vh_ref[sl, :] = vhc
        phi = _row_softmax(khc.astype(f32))
        kv_acc = kv_acc + lax.dot_general(
            phi.astype(bf16), vhc, (((0,), (0,)), ((), ())),
            preferred_element_type=f32)
        ksum = ksum + jnp.sum(phi, axis=0, keepdims=True)
        kt_blk = jnp.swapaxes(kc.reshape(CHUNK // BLK, BLK, D), 1, 2)
        kb_parts.append(jnp.sum(kt_blk, axis=-1) * (1.0 / BLK))
    kb = jnp.concatenate(kb_parts, axis=0)

    qb_parts = []
    for c in range(L // CHUNK):
        sl = slice(c * CHUNK, (c + 1) * CHUNK)
        qc = q_ref[0, 0, sl, :]
        qt_blk = jnp.swapaxes(qc.reshape(CHUNK // BLK, BLK, D), 1, 2)
        qb_parts.append(jnp.sum(qt_blk, axis=-1) * (1.0 / BLK))
    qb = jnp.concatenate(qb_parts, axis=0)

    scores = lax.dot_general(qb.astype(bf16), kb.astype(bf16),
                             (((1,), (1,)), ((), ())),
                             preferred_element_type=f32)
    col = lax.broadcasted_iota(jnp.int32, (NBLK, NBLK), 1)
    idx_cols = []
    s_work = scores
    for t in range(TK):
        m = jnp.max(s_work, axis=1, keepdims=True)
        am = jnp.min(jnp.where(s_work == m, col, NBLK), axis=1, keepdims=True)
        idx_cols.append(am)
        s_work = jnp.where(col == am, -jnp.inf, s_work)
    idx_vmem[...] = jnp.concatenate(idx_cols, axis=1)

    copy = pltpu.make_async_copy(idx_vmem, idx_smem, sem)
    copy.start()
    copy.wait()

    kv_bf = kv_acc.astype(bf16)
    w_bf = w_ref[...].astype(bf16)
    bias = b_ref[...]

    def body(i, carry):
        qraw = q_ref[0, 0, pl.ds(i * BLK, BLK), :]
        qh = qraw.astype(bf16)
        parts = []
        for t in range(TK):
            j = idx_smem[i, t]
            kt = kh_ref[pl.ds(j * BLK, BLK), :]
            st = lax.dot_general(qh, kt, (((1,), (1,)), ((), ())),
                                 preferred_element_type=f32)
            parts.append(st * SCALE)
        s = jnp.concatenate(parts, axis=1)
        m = jnp.max(s, axis=1, keepdims=True)
        p = jnp.exp(s - m)
        denom_s = jnp.sum(p, axis=1, keepdims=True)
        acc = jnp.zeros((BLK, D), dtype=f32)
        for t in range(TK):
            j = idx_smem[i, t]
            vt = vh_ref[pl.ds(j * BLK, BLK), :]
            pt = p[:, t * BLK:(t + 1) * BLK].astype(bf16)
            acc = acc + jnp.dot(pt, vt, preferred_element_type=f32)
        o_s = acc / denom_s

        phi_q = _row_softmax(qh.astype(f32))
        denom_l = 1e-05 + jnp.sum(phi_q * ksum, axis=-1, keepdims=True)
        o_l = jnp.dot(phi_q.astype(bf16), kv_bf, preferred_element_type=f32)
        o_l = o_l / denom_l
        o_l = lax.dot_general(o_l.astype(bf16), w_bf, (((1,), (1,)), ((), ())),
                              preferred_element_type=f32) + bias

        out_ref[0, 0, pl.ds(i * BLK, BLK), :] = o_s + o_l
        return carry

    lax.fori_loop(0, NBLK, body, 0)


@functools.partial(jax.jit, static_argnames=("interpret",))
def _run(q, k, v, W, b2, interpret=False):
    grid = (B, H)
    qkv_spec = pl.BlockSpec((1, 1, L, D), lambda bb, hh: (bb, hh, 0, 0))
    o = pl.pallas_call(
        _head_kernel,
        grid=grid,
        in_specs=[
            qkv_spec, qkv_spec, qkv_spec,
            pl.BlockSpec((D, D), lambda bb, hh: (0, 0)),
            pl.BlockSpec((1, D), lambda bb, hh: (0, 0)),
        ],
        out_specs=qkv_spec,
        out_shape=jax.ShapeDtypeStruct((B, H, L, D), jnp.float32),
        scratch_shapes=[
            pltpu.VMEM((L, D), jnp.bfloat16),
            pltpu.VMEM((L, D), jnp.bfloat16),
            pltpu.VMEM((NBLK, TK), jnp.int32),
            pltpu.SMEM((NBLK, TK), jnp.int32),
            pltpu.SemaphoreType.DMA,
        ],
        compiler_params=pltpu.CompilerParams(
            dimension_semantics=("parallel", "parallel"),
        ),
        interpret=interpret,
    )(q, k, v, W, b2)
    return jnp.transpose(o, (0, 2, 1, 3))


def kernel(q, k, v, W, b):
    qt = jnp.transpose(q, (0, 2, 1, 3))
    kt = jnp.transpose(k, (0, 2, 1, 3))
    vt = jnp.transpose(v, (0, 2, 1, 3))
    return _run(qt, kt, vt, W, b.reshape(1, D))


# speedup vs baseline: 1307.4460x; 1.5318x over previous
import functools

import jax
import jax.numpy as jnp
from jax import lax
from jax.experimental import pallas as pl
from jax.experimental.pallas import tpu as pltpu

B, L, H, D = 2, 2048, 16, 64
BLK = 64
NBLK = L // BLK
TK = 4
SCALE = float(D) ** -0.5
CHUNK = 256
NC = CHUNK // BLK


def _row_softmax(x):
    m = jnp.max(x, axis=-1, keepdims=True)
    e = jnp.exp(x - m)
    return e / jnp.sum(e, axis=-1, keepdims=True)


def _block_pool(xc):
    xt = jnp.swapaxes(xc.reshape(NC, BLK, D), 1, 2)
    return jnp.sum(xt, axis=-1) * (1.0 / BLK)


def _head_kernel(q_ref, k_ref, v_ref, w_ref, b_ref, out_ref,
                 kh_ref, vh_ref, qh_ref, ol_ref, idx_vmem, idx_smem, sem):
    f32 = jnp.float32
    bf16 = jnp.bfloat16

    kv_acc = jnp.zeros((D, D), dtype=f32)
    ksum = jnp.zeros((1, D), dtype=f32)
    kb_parts = []
    for c in range(L // CHUNK):
        sl = slice(c * CHUNK, (c + 1) * CHUNK)
        kc = k_ref[0, 0, sl, :]
        vc = v_ref[0, 0, sl, :]
        khc = kc.astype(bf16)
        vhc = vc.astype(bf16)
        for n in range(NC):
            kh_ref[c * NC + n] = khc[n * BLK:(n + 1) * BLK, :]
            vh_ref[c * NC + n] = vhc[n * BLK:(n + 1) * BLK, :]
        phi = _row_softmax(khc.astype(f32))
        kv_acc = kv_acc + lax.dot_general(
            phi.astype(bf16), vhc, (((0,), (0,)), ((), ())),
            preferred_element_type=f32)
        ksum = ksum + jnp.sum(phi, axis=0, keepdims=True)
        kb_parts.append(_block_pool(kc))
    kb = jnp.concatenate(kb_parts, axis=0)

    kv_bf = kv_acc.astype(bf16)
    w_bf = w_ref[...].astype(bf16)
    bias = b_ref[...]
    qb_parts = []
    for c in range(L // CHUNK):
        sl = slice(c * CHUNK, (c + 1) * CHUNK)
        qc = q_ref[0, 0, sl, :]
        qhc = qc.astype(bf16)
        for n in range(NC):
            qh_ref[c * NC + n] = qhc[n * BLK:(n + 1) * BLK, :]
        qb_parts.append(_block_pool(qc))
        phi_q = _row_softmax(qhc.astype(f32))
        denom_l = 1e-05 + jnp.sum(phi_q * ksum, axis=-1, keepdims=True)
        o_l = jnp.dot(phi_q.astype(bf16), kv_bf, preferred_element_type=f32)
        o_l = o_l / denom_l
        o_l = lax.dot_general(o_l.astype(bf16), w_bf, (((1,), (1,)), ((), ())),
                              preferred_element_type=f32) + bias
        ol_ref[sl, :] = o_l
    qb = jnp.concatenate(qb_parts, axis=0)

    scores = lax.dot_general(qb.astype(bf16), kb.astype(bf16),
                             (((1,), (1,)), ((), ())),
                             preferred_element_type=f32)
    col = lax.broadcasted_iota(jnp.int32, (NBLK, NBLK), 1)
    idx_cols = []
    s_work = scores
    for t in range(TK):
        m = jnp.max(s_work, axis=1, keepdims=True)
        am = jnp.min(jnp.where(s_work == m, col, NBLK), axis=1, keepdims=True)
        idx_cols.append(am)
        s_work = jnp.where(col == am, -jnp.inf, s_work)
    idx_vmem[...] = jnp.concatenate(idx_cols, axis=1)

    copy = pltpu.make_async_copy(idx_vmem, idx_smem, sem)
    copy.start()
    copy.wait()

    for i in range(NBLK):
        qh = qh_ref[i]
        kcat = jnp.concatenate(
            [kh_ref[idx_smem[i, t]] for t in range(TK)], axis=0)
        vcat = jnp.concatenate(
            [vh_ref[idx_smem[i, t]] for t in range(TK)], axis=0)
        s = lax.dot_general(qh, kcat, (((1,), (1,)), ((), ())),
                            preferred_element_type=f32) * SCALE
        m = jnp.max(s, axis=1, keepdims=True)
        p = jnp.exp(s - m)
        denom_s = jnp.sum(p, axis=1, keepdims=True)
        o_s = jnp.dot(p.astype(bf16), vcat, preferred_element_type=f32)
        o_s = o_s / denom_s
        out_ref[0, 0, i * BLK:(i + 1) * BLK, :] = (
            o_s + ol_ref[i * BLK:(i + 1) * BLK, :])


@functools.partial(jax.jit, static_argnames=("interpret",))
def _run(q, k, v, W, b2, interpret=False):
    grid = (B, H)
    qkv_spec = pl.BlockSpec((1, 1, L, D), lambda bb, hh: (bb, hh, 0, 0))
    o = pl.pallas_call(
        _head_kernel,
        grid=grid,
        in_specs=[
            qkv_spec, qkv_spec, qkv_spec,
            pl.BlockSpec((D, D), lambda bb, hh: (0, 0)),
            pl.BlockSpec((1, D), lambda bb, hh: (0, 0)),
        ],
        out_specs=qkv_spec,
        out_shape=jax.ShapeDtypeStruct((B, H, L, D), jnp.float32),
        scratch_shapes=[
            pltpu.VMEM((NBLK, BLK, D), jnp.bfloat16),
            pltpu.VMEM((NBLK, BLK, D), jnp.bfloat16),
            pltpu.VMEM((NBLK, BLK, D), jnp.bfloat16),
            pltpu.VMEM((L, D), jnp.float32),
            pltpu.VMEM((NBLK, TK), jnp.int32),
            pltpu.SMEM((NBLK, TK), jnp.int32),
            pltpu.SemaphoreType.DMA,
        ],
        compiler_params=pltpu.CompilerParams(
            dimension_semantics=("parallel", "parallel"),
        ),
        interpret=interpret,
    )(q, k, v, W, b2)
    return jnp.transpose(o, (0, 2, 1, 3))


def kernel(q, k, v, W, b):
    qt = jnp.transpose(q, (0, 2, 1, 3))
    kt = jnp.transpose(k, (0, 2, 1, 3))
    vt = jnp.transpose(v, (0, 2, 1, 3))
    return _run(qt, kt, vt, W, b.reshape(1, D))


# speedup vs baseline: 1895.0161x; 1.4494x over previous
import functools

import jax
import jax.numpy as jnp
from jax import lax
from jax.experimental import pallas as pl
from jax.experimental.pallas import tpu as pltpu

B, L, H, D = 2, 2048, 16, 64
BLK = 64
NBLK = L // BLK
TK = 4
SCALE = float(D) ** -0.5
CHUNK = 256
NC = CHUNK // BLK


def _row_softmax(x):
    m = jnp.max(x, axis=-1, keepdims=True)
    e = jnp.exp(x - m)
    return e / jnp.sum(e, axis=-1, keepdims=True)


def _block_pool(xc):
    xt = jnp.swapaxes(xc.reshape(NC, BLK, D), 1, 2)
    return jnp.sum(xt, axis=-1) * (1.0 / BLK)


def _head_kernel(q_ref, k_ref, v_ref, w_ref, b_ref, out_ref,
                 kh_ref, vh_ref, qh_ref, ol_ref, s_ref, p_ref, l_ref,
                 idx_vmem, idx_smem, sem):
    f32 = jnp.float32
    bf16 = jnp.bfloat16

    kv_acc = jnp.zeros((D, D), dtype=f32)
    ksum = jnp.zeros((1, D), dtype=f32)
    kb_parts = []
    for c in range(L // CHUNK):
        sl = slice(c * CHUNK, (c + 1) * CHUNK)
        kc = k_ref[0, 0, sl, :]
        vc = v_ref[0, 0, sl, :]
        khc = kc.astype(bf16)
        vhc = vc.astype(bf16)
        for n in range(NC):
            kh_ref[c * NC + n] = khc[n * BLK:(n + 1) * BLK, :]
            vh_ref[c * NC + n] = vhc[n * BLK:(n + 1) * BLK, :]
        phi = _row_softmax(khc.astype(f32))
        kv_acc = kv_acc + lax.dot_general(
            phi.astype(bf16), vhc, (((0,), (0,)), ((), ())),
            preferred_element_type=f32)
        ksum = ksum + jnp.sum(phi, axis=0, keepdims=True)
        kb_parts.append(_block_pool(kc))
    kb = jnp.concatenate(kb_parts, axis=0)

    kv_bf = kv_acc.astype(bf16)
    w_bf = w_ref[...].astype(bf16)
    bias = b_ref[...]
    qb_parts = []
    for c in range(L // CHUNK):
        sl = slice(c * CHUNK, (c + 1) * CHUNK)
        qc = q_ref[0, 0, sl, :]
        qhc = qc.astype(bf16)
        qsc = qhc * jnp.asarray(SCALE, bf16)
        for n in range(NC):
            qh_ref[c * NC + n] = qsc[n * BLK:(n + 1) * BLK, :]
        qb_parts.append(_block_pool(qc))
        phi_q = _row_softmax(qhc.astype(f32))
        denom_l = 1e-05 + jnp.sum(phi_q * ksum, axis=-1, keepdims=True)
        o_l = jnp.dot(phi_q.astype(bf16), kv_bf, preferred_element_type=f32)
        o_l = o_l / denom_l
        o_l = lax.dot_general(o_l.astype(bf16), w_bf, (((1,), (1,)), ((), ())),
                              preferred_element_type=f32) + bias
        ol_ref[sl, :] = o_l
    qb = jnp.concatenate(qb_parts, axis=0)

    scores = lax.dot_general(qb.astype(bf16), kb.astype(bf16),
                             (((1,), (1,)), ((), ())),
                             preferred_element_type=f32)
    col = lax.broadcasted_iota(jnp.int32, (NBLK, NBLK), 1)
    idx_cols = []
    s_work = scores
    for t in range(TK):
        m = jnp.max(s_work, axis=1, keepdims=True)
        am = jnp.min(jnp.where(s_work == m, col, NBLK), axis=1, keepdims=True)
        idx_cols.append(am)
        s_work = jnp.where(col == am, -jnp.inf, s_work)
    idx_vmem[...] = jnp.concatenate(idx_cols, axis=1)

    copy = pltpu.make_async_copy(idx_vmem, idx_smem, sem)
    copy.start()
    copy.wait()

    for i in range(NBLK):
        kcat = jnp.concatenate(
            [kh_ref[idx_smem[i, t]] for t in range(TK)], axis=0)
        s_ref[i * BLK:(i + 1) * BLK, :] = lax.dot_general(
            qh_ref[i], kcat, (((1,), (1,)), ((), ())),
            preferred_element_type=f32)

    SMC = 512
    for c in range(L // SMC):
        sl = slice(c * SMC, (c + 1) * SMC)
        s = s_ref[sl, :]
        m = jnp.max(s, axis=1, keepdims=True)
        p = jnp.exp(s - m)
        l_ref[sl, :] = jnp.sum(p, axis=1, keepdims=True)
        p_ref[sl, :] = p.astype(bf16)

    for i in range(NBLK):
        vcat = jnp.concatenate(
            [vh_ref[idx_smem[i, t]] for t in range(TK)], axis=0)
        sl = slice(i * BLK, (i + 1) * BLK)
        o_s = jnp.dot(p_ref[sl, :], vcat, preferred_element_type=f32)
        o_s = o_s / l_ref[sl, :]
        out_ref[0, 0, sl, :] = o_s + ol_ref[sl, :]


@functools.partial(jax.jit, static_argnames=("interpret",))
def _run(q, k, v, W, b2, interpret=False):
    grid = (B, H)
    qkv_spec = pl.BlockSpec((1, 1, L, D), lambda bb, hh: (bb, hh, 0, 0))
    o = pl.pallas_call(
        _head_kernel,
        grid=grid,
        in_specs=[
            qkv_spec, qkv_spec, qkv_spec,
            pl.BlockSpec((D, D), lambda bb, hh: (0, 0)),
            pl.BlockSpec((1, D), lambda bb, hh: (0, 0)),
        ],
        out_specs=qkv_spec,
        out_shape=jax.ShapeDtypeStruct((B, H, L, D), jnp.float32),
        scratch_shapes=[
            pltpu.VMEM((NBLK, BLK, D), jnp.bfloat16),
            pltpu.VMEM((NBLK, BLK, D), jnp.bfloat16),
            pltpu.VMEM((NBLK, BLK, D), jnp.bfloat16),
            pltpu.VMEM((L, D), jnp.float32),
            pltpu.VMEM((L, TK * BLK), jnp.float32),
            pltpu.VMEM((L, TK * BLK), jnp.bfloat16),
            pltpu.VMEM((L, 1), jnp.float32),
            pltpu.VMEM((NBLK, TK), jnp.int32),
            pltpu.SMEM((NBLK, TK), jnp.int32),
            pltpu.SemaphoreType.DMA,
        ],
        compiler_params=pltpu.CompilerParams(
            dimension_semantics=("parallel", "parallel"),
        ),
        interpret=interpret,
    )(q, k, v, W, b2)
    return jnp.transpose(o, (0, 2, 1, 3))


def kernel(q, k, v, W, b):
    qt = jnp.transpose(q, (0, 2, 1, 3))
    kt = jnp.transpose(k, (0, 2, 1, 3))
    vt = jnp.transpose(v, (0, 2, 1, 3))
    return _run(qt, kt, vt, W, b.reshape(1, D))
